# baseline (speedup 1.0000x reference)
import jax
import jax.numpy as jnp
from jax.experimental import pallas as pl
from jax.experimental.pallas import tpu as pltpu

_RR = 200


def _add_kernel(x_ref, pe_ref, o_ref):
    o_ref[...] = x_ref[...] + pe_ref[...]


def kernel(x, encoding):
    B, L, D = x.shape
    LD = L * D
    x2 = x.transpose(1, 2, 0).reshape(LD, B)
    pe = encoding[:L].reshape(LD, 1)
    grid = (LD // _RR,)
    out = pl.pallas_call(
        _add_kernel,
        grid=grid,
        in_specs=[
            pl.BlockSpec((_RR, B), lambda i: (i, 0)),
            pl.BlockSpec((_RR, 1), lambda i: (i, 0)),
        ],
        out_specs=pl.BlockSpec((_RR, B), lambda i: (i, 0)),
        out_shape=jax.ShapeDtypeStruct((LD, B), x.dtype),
        compiler_params=pltpu.CompilerParams(
            dimension_semantics=("parallel",),
            vmem_limit_bytes=110 * 1024 * 1024,
        ),
    )(x2, pe)
    return out.reshape(L, D, B).transpose(2, 0, 1)


# speedup vs baseline: 1.0217x; 1.0217x over previous
import jax
import jax.numpy as jnp
from jax.experimental import pallas as pl
from jax.experimental.pallas import tpu as pltpu

_RR = 256
_K = 6


def _make_body(num_chunks):
    def body(x_hbm, pe_vmem, o_hbm, in_buf, out_buf, in_sem, out_sem):
        def start_in(i):
            s = i % _K
            pltpu.make_async_copy(
                x_hbm.at[pl.ds(i * _RR, _RR), :], in_buf.at[s], in_sem.at[s]
            ).start()

        for i in range(min(_K, num_chunks)):
            start_in(i)
        for i in range(num_chunks):
            s = i % _K
            pltpu.make_async_copy(
                x_hbm.at[pl.ds(i * _RR, _RR), :], in_buf.at[s], in_sem.at[s]
            ).wait()
            if i >= _K:
                pltpu.make_async_copy(
                    out_buf.at[s],
                    o_hbm.at[pl.ds((i - _K) * _RR, _RR), :],
                    out_sem.at[s],
                ).wait()
            out_buf[s] = in_buf[s] + pe_vmem[pl.ds(i * _RR, _RR), :]
            pltpu.make_async_copy(
                out_buf.at[s], o_hbm.at[pl.ds(i * _RR, _RR), :], out_sem.at[s]
            ).start()
            if i + _K < num_chunks:
                start_in(i + _K)
        for i in range(max(0, num_chunks - _K), num_chunks):
            s = i % _K
            pltpu.make_async_copy(
                out_buf.at[s], o_hbm.at[pl.ds(i * _RR, _RR), :], out_sem.at[s]
            ).wait()

    return body


def kernel(x, encoding):
    B, L, D = x.shape
    LD = L * D
    num_chunks = LD // _RR
    x2 = x.transpose(1, 2, 0).reshape(LD, B)
    pe = encoding[:L].reshape(LD, 1)
    out = pl.pallas_call(
        _make_body(num_chunks),
        in_specs=[
            pl.BlockSpec(memory_space=pl.ANY),
            pl.BlockSpec(memory_space=pltpu.VMEM),
        ],
        out_specs=pl.BlockSpec(memory_space=pl.ANY),
        out_shape=jax.ShapeDtypeStruct((LD, B), x.dtype),
        scratch_shapes=[
            pltpu.VMEM((_K, _RR, B), x.dtype),
            pltpu.VMEM((_K, _RR, B), x.dtype),
            pltpu.SemaphoreType.DMA((_K,)),
            pltpu.SemaphoreType.DMA((_K,)),
        ],
        compiler_params=pltpu.CompilerParams(
            vmem_limit_bytes=110 * 1024 * 1024,
        ),
    )(x2, pe)
    return out.reshape(L, D, B).transpose(2, 0, 1)


# speedup vs baseline: 1.1195x; 1.0957x over previous
import jax
import jax.numpy as jnp
from jax.experimental import pallas as pl
from jax.experimental.pallas import tpu as pltpu

_CL = 8
_K = 3


def _make_body(L, num_chunks):
    def body(x_hbm, et_vmem, o_hbm, in_buf, out_buf, in_sem, out_sem):
        def start_in(c):
            s = c % _K
            pltpu.make_async_copy(
                x_hbm.at[pl.ds(c * _CL, _CL)], in_buf.at[s], in_sem.at[s]
            ).start()

        for c in range(min(_K, num_chunks)):
            start_in(c)
        for c in range(num_chunks):
            s = c % _K
            pltpu.make_async_copy(
                x_hbm.at[pl.ds(c * _CL, _CL)], in_buf.at[s], in_sem.at[s]
            ).wait()
            if c >= _K:
                pltpu.make_async_copy(
                    out_buf.at[s],
                    o_hbm.at[pl.ds((c - _K) * _CL, _CL)],
                    out_sem.at[s],
                ).wait()
            for j in range(_CL):
                l = c * _CL + j
                out_buf[s, j] = in_buf[s, j] + et_vmem[:, l:l + 1]
            pltpu.make_async_copy(
                out_buf.at[s], o_hbm.at[pl.ds(c * _CL, _CL)], out_sem.at[s]
            ).start()
            if c + _K < num_chunks:
                start_in(c + _K)
        for c in range(max(0, num_chunks - _K), num_chunks):
            s = c % _K
            pltpu.make_async_copy(
                out_buf.at[s], o_hbm.at[pl.ds(c * _CL, _CL)], out_sem.at[s]
            ).wait()

    return body


def kernel(x, encoding):
    B, L, D = x.shape
    num_chunks = L // _CL
    x3 = x.transpose(1, 2, 0)
    et = encoding.T
    out = pl.pallas_call(
        _make_body(L, num_chunks),
        in_specs=[
            pl.BlockSpec(memory_space=pl.ANY),
            pl.BlockSpec(memory_space=pltpu.VMEM),
        ],
        out_specs=pl.BlockSpec(memory_space=pl.ANY),
        out_shape=jax.ShapeDtypeStruct((L, D, B), x.dtype),
        scratch_shapes=[
            pltpu.VMEM((_K, _CL, D, B), x.dtype),
            pltpu.VMEM((_K, _CL, D, B), x.dtype),
            pltpu.SemaphoreType.DMA((_K,)),
            pltpu.SemaphoreType.DMA((_K,)),
        ],
        compiler_params=pltpu.CompilerParams(
            vmem_limit_bytes=110 * 1024 * 1024,
        ),
    )(x3, et)
    return out.transpose(2, 0, 1)
